# Initial kernel scaffold; baseline (speedup 1.0000x reference)
import jax
import jax.numpy as jnp
from jax.experimental import pallas as pl


def kernel(time_delta, event_mask, sin_div_term, cos_div_term):
    raise NotImplementedError("write your pallas kernel here")


# speedup vs baseline: 2.4397x; 2.4397x over previous
import math

import numpy as np

import jax
import jax.numpy as jnp
from jax.experimental import pallas as pl
from jax.experimental.pallas import tpu as pltpu

_E = 64
_LANES = 128
_L = _LANES * _E

_D0 = 6.27928880709451
_D1 = -41.117128993718936
_D2 = 78.1313235840818
_D3 = -56.565643936619566


def _pe_kernel(t_ref, divt_ref, ptw_ref, out_ref):
    tm = t_ref.shape[0]
    t = t_ref[...]
    lane = jax.lax.broadcasted_iota(jnp.int32, (tm, _LANES), 1)
    base = (lane >= _E).astype(jnp.int32)
    pieces = []
    for a in range(_LANES // 2):
        pieces.append(jnp.take_along_axis(t, base + (2 * a), axis=1))
    texp = jnp.concatenate(pieces, axis=1)
    r = texp * divt_ref[...] + ptw_ref[...]
    k = jax.lax.round(r, jax.lax.RoundingMethod.TO_NEAREST_EVEN)
    g = r - k
    u = g * g
    p = _D3
    p = p * u + _D2
    p = p * u + _D1
    p = p * u + _D0
    out_ref[...] = p * g


def kernel(time_delta, event_mask, sin_div_term, cos_div_term):
    td = jnp.where(event_mask, time_delta, jnp.zeros_like(time_delta))
    t = jnp.cumsum(td, axis=-1, dtype=jnp.float32)
    t = jnp.concatenate([jnp.zeros_like(t[:, :1]), t[:, :-1]], axis=-1)

    B, S = t.shape
    N = B * S
    M = N // _LANES
    t_lane = t.reshape(M, _LANES)

    div_full = jnp.zeros((_E,), jnp.float32)
    div_full = div_full.at[0::2].set(sin_div_term.astype(jnp.float32))
    div_full = div_full.at[1::2].set(cos_div_term.astype(jnp.float32))
    div_turns = div_full * jnp.float32(1.0 / (2.0 * math.pi))
    pturns = jnp.zeros((_E,), jnp.float32).at[1::2].set(0.25)

    divw = jnp.tile(div_turns, _LANES).reshape(1, _L)
    ptw = jnp.tile(pturns, _LANES).reshape(1, _L)

    tile_m = 128
    grid = (M // tile_m,)

    cost = pl.CostEstimate(
        flops=20 * N * _E,
        transcendentals=0,
        bytes_accessed=N * 4 + N * _E * 4 + 3 * _L * 4,
    )

    out_wide = pl.pallas_call(
        _pe_kernel,
        out_shape=jax.ShapeDtypeStruct((M, _L), jnp.float32),
        grid=grid,
        in_specs=[
            pl.BlockSpec((tile_m, _LANES), lambda i: (i, 0)),
            pl.BlockSpec((1, _L), lambda i: (0, 0)),
            pl.BlockSpec((1, _L), lambda i: (0, 0)),
        ],
        out_specs=pl.BlockSpec((tile_m, _L), lambda i: (i, 0)),
        compiler_params=pltpu.CompilerParams(
            dimension_semantics=("parallel",)),
        cost_estimate=cost,
    )(t_lane, divw, ptw)

    return out_wide.reshape(B, S, _E)


# speedup vs baseline: 2.9304x; 1.2011x over previous
import math

import jax
import jax.numpy as jnp
from jax.experimental import pallas as pl
from jax.experimental.pallas import tpu as pltpu

_E = 64

_D0 = 6.27928880709451
_D1 = -41.117128993718936
_D2 = 78.1313235840818
_D3 = -56.565643936619566


def _pe_kernel(tT_ref, divt_ref, ptw_ref, out_ref):
    ts, tb = tT_ref.shape
    tT = tT_ref[...]
    dv = divt_ref[...]
    pt = ptw_ref[...]
    for b in range(tb):
        idx = jnp.full((ts, _E), b, jnp.int32)
        tcol = jnp.take_along_axis(tT, idx, axis=1)
        r = tcol * dv + pt
        k = jax.lax.round(r, jax.lax.RoundingMethod.TO_NEAREST_EVEN)
        g = r - k
        u = g * g
        p = _D3
        p = p * u + _D2
        p = p * u + _D1
        p = p * u + _D0
        out_ref[b] = p * g


def kernel(time_delta, event_mask, sin_div_term, cos_div_term):
    td = jnp.where(event_mask, time_delta, jnp.zeros_like(time_delta))
    t = jnp.cumsum(td, axis=-1, dtype=jnp.float32)
    t = jnp.concatenate([jnp.zeros_like(t[:, :1]), t[:, :-1]], axis=-1)

    B, S = t.shape
    tT = t.T

    div_full = jnp.zeros((_E,), jnp.float32)
    div_full = div_full.at[0::2].set(sin_div_term.astype(jnp.float32))
    div_full = div_full.at[1::2].set(cos_div_term.astype(jnp.float32))
    divt = (div_full * jnp.float32(1.0 / (2.0 * math.pi))).reshape(1, _E)
    ptw = jnp.zeros((_E,), jnp.float32).at[1::2].set(0.25).reshape(1, _E)

    tile_b = 128
    tile_s = 64
    grid = (B // tile_b, S // tile_s)

    cost = pl.CostEstimate(
        flops=14 * B * S * _E,
        transcendentals=0,
        bytes_accessed=B * S * 4 + B * S * _E * 4 + 2 * _E * 4,
    )

    out = pl.pallas_call(
        _pe_kernel,
        out_shape=jax.ShapeDtypeStruct((B, S, _E), jnp.float32),
        grid=grid,
        in_specs=[
            pl.BlockSpec((tile_s, tile_b), lambda i, j: (j, i)),
            pl.BlockSpec((1, _E), lambda i, j: (0, 0)),
            pl.BlockSpec((1, _E), lambda i, j: (0, 0)),
        ],
        out_specs=pl.BlockSpec((tile_b, tile_s, _E), lambda i, j: (i, j, 0)),
        compiler_params=pltpu.CompilerParams(
            dimension_semantics=("parallel", "parallel")),
        cost_estimate=cost,
    )(tT, divt, ptw)

    return out


# speedup vs baseline: 3.0045x; 1.0253x over previous
import math

import jax
import jax.numpy as jnp
from jax.experimental import pallas as pl
from jax.experimental.pallas import tpu as pltpu

_E = 64

_D0 = 6.27928880709451
_D1 = -41.117128993718936
_D2 = 78.1313235840818
_D3 = -56.565643936619566


def _pe_kernel(t_ref, divt_ref, ptw_ref, out_ref):
    tb, ts = t_ref.shape
    tT = jnp.transpose(t_ref[...])
    dv = divt_ref[...]
    pt = ptw_ref[...]
    for b in range(tb):
        idx = jnp.full((ts, _E), b, jnp.int32)
        tcol = jnp.take_along_axis(tT, idx, axis=1)
        r = tcol * dv + pt
        k = jax.lax.round(r, jax.lax.RoundingMethod.TO_NEAREST_EVEN)
        g = r - k
        u = g * g
        p = _D3
        p = p * u + _D2
        p = p * u + _D1
        p = p * u + _D0
        out_ref[b] = p * g


def kernel(time_delta, event_mask, sin_div_term, cos_div_term):
    td = jnp.where(event_mask, time_delta, jnp.zeros_like(time_delta))
    t = jnp.cumsum(td, axis=-1, dtype=jnp.float32)
    t = jnp.concatenate([jnp.zeros_like(t[:, :1]), t[:, :-1]], axis=-1)

    B, S = t.shape

    div_full = jnp.zeros((_E,), jnp.float32)
    div_full = div_full.at[0::2].set(sin_div_term.astype(jnp.float32))
    div_full = div_full.at[1::2].set(cos_div_term.astype(jnp.float32))
    divt = (div_full * jnp.float32(1.0 / (2.0 * math.pi))).reshape(1, _E)
    ptw = jnp.zeros((_E,), jnp.float32).at[1::2].set(0.25).reshape(1, _E)

    tile_b = 128
    tile_s = 128
    grid = (B // tile_b, S // tile_s)

    cost = pl.CostEstimate(
        flops=14 * B * S * _E,
        transcendentals=0,
        bytes_accessed=B * S * 4 + B * S * _E * 4 + 2 * _E * 4,
    )

    out = pl.pallas_call(
        _pe_kernel,
        out_shape=jax.ShapeDtypeStruct((B, S, _E), jnp.float32),
        grid=grid,
        in_specs=[
            pl.BlockSpec((tile_b, tile_s), lambda i, j: (i, j)),
            pl.BlockSpec((1, _E), lambda i, j: (0, 0)),
            pl.BlockSpec((1, _E), lambda i, j: (0, 0)),
        ],
        out_specs=pl.BlockSpec((tile_b, tile_s, _E), lambda i, j: (i, j, 0)),
        compiler_params=pltpu.CompilerParams(
            dimension_semantics=("parallel", "parallel")),
        cost_estimate=cost,
    )(t, divt, ptw)

    return out


# speedup vs baseline: 7.6199x; 2.5361x over previous
import math

import jax
import jax.numpy as jnp
from jax.experimental import pallas as pl
from jax.experimental.pallas import tpu as pltpu

_E = 64

_D0 = 6.27928880709451
_D1 = -41.117128993718936
_D2 = 78.1313235840818
_D3 = -56.565643936619566


def _pe_kernel(t_ref, dv_ref, pt_ref, out_ref):
    tb, ts = t_ref.shape
    dv = dv_ref[...]
    pt = pt_ref[...]
    for b in range(tb):
        x = jnp.broadcast_to(t_ref[b : b + 1, :], (_E, ts))
        r = x * dv + pt
        k = jax.lax.round(r, jax.lax.RoundingMethod.TO_NEAREST_EVEN)
        g = r - k
        u = g * g
        p = _D3
        p = p * u + _D2
        p = p * u + _D1
        p = p * u + _D0
        out_ref[b] = p * g


def kernel(time_delta, event_mask, sin_div_term, cos_div_term):
    td = jnp.where(event_mask, time_delta, jnp.zeros_like(time_delta))
    t = jnp.cumsum(td, axis=-1, dtype=jnp.float32)
    t = jnp.concatenate([jnp.zeros_like(t[:, :1]), t[:, :-1]], axis=-1)

    B, S = t.shape

    div_full = jnp.zeros((_E,), jnp.float32)
    div_full = div_full.at[0::2].set(sin_div_term.astype(jnp.float32))
    div_full = div_full.at[1::2].set(cos_div_term.astype(jnp.float32))
    div_turns = div_full * jnp.float32(1.0 / (2.0 * math.pi))
    pturns = jnp.zeros((_E,), jnp.float32).at[1::2].set(0.25)

    tile_b = 8
    tile_s = 512
    grid = (B // tile_b, S // tile_s)

    dvb = jnp.broadcast_to(div_turns.reshape(_E, 1), (_E, tile_s))
    ptb = jnp.broadcast_to(pturns.reshape(_E, 1), (_E, tile_s))

    cost = pl.CostEstimate(
        flops=12 * B * S * _E,
        transcendentals=0,
        bytes_accessed=B * S * 4 + B * S * _E * 4 + 2 * _E * tile_s * 4,
    )

    out_t = pl.pallas_call(
        _pe_kernel,
        out_shape=jax.ShapeDtypeStruct((B, _E, S), jnp.float32),
        grid=grid,
        in_specs=[
            pl.BlockSpec((tile_b, tile_s), lambda i, j: (i, j)),
            pl.BlockSpec((_E, tile_s), lambda i, j: (0, 0)),
            pl.BlockSpec((_E, tile_s), lambda i, j: (0, 0)),
        ],
        out_specs=pl.BlockSpec((tile_b, _E, tile_s), lambda i, j: (i, 0, j)),
        compiler_params=pltpu.CompilerParams(
            dimension_semantics=("parallel", "parallel")),
        cost_estimate=cost,
    )(t, dvb, ptb)

    return jnp.swapaxes(out_t, 1, 2)


# speedup vs baseline: 11.8650x; 1.5571x over previous
import math

import jax
import jax.numpy as jnp
from jax.experimental import pallas as pl
from jax.experimental.pallas import tpu as pltpu

_E = 64

_D0 = 6.27928880709451
_D1 = -41.117128993718936
_D2 = 78.1313235840818
_D3 = -56.565643936619566


def _pe_kernel(td_ref, m_ref, dv_ref, pt_ref, out_ref):
    tb, s = td_ref.shape
    x = jnp.where(m_ref[...], td_ref[...], jnp.zeros_like(td_ref))
    t = jnp.concatenate([jnp.zeros((tb, 1), jnp.float32), x[:, : s - 1]],
                        axis=1)
    k = 1
    while k < s:
        t = t + jnp.concatenate(
            [jnp.zeros((tb, k), jnp.float32), t[:, : s - k]], axis=1)
        k *= 2
    dv = dv_ref[...]
    pt = pt_ref[...]
    for b in range(tb):
        xb = jnp.broadcast_to(t[b : b + 1, :], (_E, s))
        r = xb * dv + pt
        k = jax.lax.round(r, jax.lax.RoundingMethod.TO_NEAREST_EVEN)
        g = r - k
        u = g * g
        p = _D3
        p = p * u + _D2
        p = p * u + _D1
        p = p * u + _D0
        out_ref[b] = p * g


def kernel(time_delta, event_mask, sin_div_term, cos_div_term):
    B, S = time_delta.shape

    div_full = jnp.zeros((_E,), jnp.float32)
    div_full = div_full.at[0::2].set(sin_div_term.astype(jnp.float32))
    div_full = div_full.at[1::2].set(cos_div_term.astype(jnp.float32))
    div_turns = div_full * jnp.float32(1.0 / (2.0 * math.pi))
    pturns = jnp.zeros((_E,), jnp.float32).at[1::2].set(0.25)

    dvb = jnp.broadcast_to(div_turns.reshape(_E, 1), (_E, S))
    ptb = jnp.broadcast_to(pturns.reshape(_E, 1), (_E, S))

    tile_b = 8
    grid = (B // tile_b,)

    cost = pl.CostEstimate(
        flops=13 * B * S * _E,
        transcendentals=0,
        bytes_accessed=2 * B * S * 4 + B * S * _E * 4 + 2 * _E * S * 4,
    )

    out_t = pl.pallas_call(
        _pe_kernel,
        out_shape=jax.ShapeDtypeStruct((B, _E, S), jnp.float32),
        grid=grid,
        in_specs=[
            pl.BlockSpec((tile_b, S), lambda i: (i, 0)),
            pl.BlockSpec((tile_b, S), lambda i: (i, 0)),
            pl.BlockSpec((_E, S), lambda i: (0, 0)),
            pl.BlockSpec((_E, S), lambda i: (0, 0)),
        ],
        out_specs=pl.BlockSpec((tile_b, _E, S), lambda i: (i, 0, 0)),
        compiler_params=pltpu.CompilerParams(
            dimension_semantics=("parallel",)),
        cost_estimate=cost,
    )(time_delta, event_mask, dvb, ptb)

    return jnp.swapaxes(out_t, 1, 2)
